# baseline (speedup 1.0000x reference)
import functools

import jax
import jax.numpy as jnp
from jax import lax
from jax.experimental import pallas as pl
from jax.experimental.pallas import tpu as pltpu
from jax.experimental.pallas import tpu_sc as plsc

N = 10000
E = 320000
D = 128
P = 2

NC = 2
NS = 16
CHUNK = 128
SUP = 16
NSUP = -(-E // (NS * SUP * CHUNK))
TPT = NSUP * SUP * CHUNK
E_PAD = NS * TPT

BN = 1000
NB = N // BN


def _tc1_body(feats_ref, wft_ref, bf_ref, w0t_ref, w1t_ref, out_ref):
    x = feats_ref[...]
    h = jnp.dot(x, wft_ref[...], preferred_element_type=jnp.float32)
    h = h + bf_ref[...]
    h = jnp.where(h > 0, h, jnp.exp(jnp.minimum(h, 0.0)) - 1.0)
    out_ref[0] = jnp.dot(h, w0t_ref[...], preferred_element_type=jnp.float32)
    out_ref[1] = jnp.dot(h, w1t_ref[...], preferred_element_type=jnp.float32)


def _tc1(feats, wft, bf, w0t, w1t):
    return pl.pallas_call(
        _tc1_body,
        grid=(NB,),
        in_specs=[
            pl.BlockSpec((BN, D), lambda i: (i, 0)),
            pl.BlockSpec((D, D), lambda i: (0, 0)),
            pl.BlockSpec((1, D), lambda i: (0, 0)),
            pl.BlockSpec((D, D), lambda i: (0, 0)),
            pl.BlockSpec((D, D), lambda i: (0, 0)),
        ],
        out_specs=pl.BlockSpec((P, BN, D), lambda i: (0, i, 0)),
        out_shape=jax.ShapeDtypeStruct((P, N, D), jnp.float32),
    )(feats, wft, bf, w0t, w1t)


def _sc_body(seq_hbm, src_hbm, dst_hbm, val_hbm, out_hbm,
             idx_src, idx_dst, vals_v, rows, acc, sem, sem2, ssem):
    cid = lax.axis_index("c")
    sid = lax.axis_index("s")

    def _zrow(r, _):
        for j in range(D // 16):
            rows[0, r, pl.ds(j * 16, 16)] = jnp.zeros((16,), jnp.float32)
        return 0
    lax.fori_loop(0, CHUNK, _zrow, 0)

    base = sid * 624
    for i in range(4):
        pltpu.sync_copy(rows.at[0], acc.at[pl.ds(base + i * 128, 128)])
    pltpu.sync_copy(rows.at[0, pl.ds(0, 112)],
                    acc.at[pl.ds(base + 512, 112)])

    @pl.when(sid == 0)
    def _():
        pltpu.sync_copy(rows.at[0, pl.ds(0, 16)],
                        acc.at[pl.ds(NS * 624, 16)])
    plsc.subcore_barrier()

    def _scale(slot, c):
        def _grp(g, _):
            vv = vals_v[c, pl.ds(g * 16, 16)]
            for k in range(16):
                e = g * 16 + k
                v = vv[k]
                for j in range(D // 16):
                    sl = pl.ds(j * 16, 16)
                    rows[slot, e, sl] = rows[slot, e, sl] * v
            return 0
        lax.fori_loop(0, CHUNK // 16, _grp, 0)

    def _gather_start(c, slot, gsem):
        pltpu.async_copy(seq_hbm.at[idx_src.at[c]], rows.at[slot], gsem)

    def _gather_wait(c, slot, gsem):
        pltpu.make_async_copy(seq_hbm.at[idx_src.at[c]], rows.at[slot],
                              gsem).wait()

    def _sup(s, _):
        pltpu.sync_copy(src_hbm.at[cid, sid, s], idx_src)
        pltpu.sync_copy(dst_hbm.at[cid, sid, s], idx_dst)
        pltpu.sync_copy(val_hbm.at[cid, sid, s], vals_v)

        _gather_start(0, 0, sem)

        def _step(j, _):
            c0 = 2 * j
            c1 = 2 * j + 1
            _gather_wait(c0, 0, sem)
            _gather_start(c1, 1, sem2)
            _scale(0, c0)
            pltpu.sync_copy(rows.at[0], acc.at[idx_dst.at[c0]], add=True)

            _gather_wait(c1, 1, sem2)

            @pl.when(j < SUP // 2 - 1)
            def _():
                _gather_start(c1 + 1, 0, sem)
            _scale(1, c1)
            pltpu.sync_copy(rows.at[1], acc.at[idx_dst.at[c1]], add=True)
            return 0

        lax.fori_loop(0, SUP // 2, _step, 0)
        return 0

    lax.fori_loop(0, NSUP, _sup, 0)
    plsc.subcore_barrier()

    out_base = cid * N + base
    pltpu.sync_copy(acc.at[pl.ds(base, 624)],
                    out_hbm.at[pl.ds(out_base, 624)])

    @pl.when(sid == 0)
    def _():
        pltpu.sync_copy(acc.at[pl.ds(NS * 624, 16)],
                        out_hbm.at[pl.ds(cid * N + NS * 624, 16)])


@functools.partial(
    pl.kernel,
    out_type=jax.ShapeDtypeStruct((P * N, D), jnp.float32),
    mesh=plsc.VectorSubcoreMesh(core_axis_name="c", subcore_axis_name="s"),
    scratch_types=[
        pltpu.VMEM((SUP, CHUNK), jnp.int32),
        pltpu.VMEM((SUP, CHUNK), jnp.int32),
        pltpu.VMEM((SUP, CHUNK), jnp.float32),
        pltpu.VMEM((2, CHUNK, D), jnp.float32),
        pltpu.VMEM_SHARED((N, D), jnp.float32),
        pltpu.SemaphoreType.DMA,
        pltpu.SemaphoreType.DMA,
        pltpu.SemaphoreType.DMA,
    ],
)
def _sc_aggregate(seq_hbm, src_hbm, dst_hbm, val_hbm, out_hbm,
                  idx_src, idx_dst, vals_v, rows, acc, sem, sem2, ssem):
    _sc_body(seq_hbm, src_hbm, dst_hbm, val_hbm, out_hbm,
             idx_src, idx_dst, vals_v, rows, acc, sem, sem2, ssem)


def _prelu(sums, bg, ag):
    x = sums + bg
    return jnp.where(x >= 0, x, ag * x)


def _tc2a_body(sums_ref, bg_ref, ag_ref, wat_ref, ba_ref, tsum_ref):
    i = pl.program_id(0)
    e = _prelu(sums_ref[...], bg_ref[...], ag_ref[...])
    t0 = jnp.tanh(jnp.dot(e[0], wat_ref[...],
                          preferred_element_type=jnp.float32) + ba_ref[...])
    t1 = jnp.tanh(jnp.dot(e[1], wat_ref[...],
                          preferred_element_type=jnp.float32) + ba_ref[...])
    part = jnp.stack([jnp.sum(t0, axis=0), jnp.sum(t1, axis=0)])

    @pl.when(i == 0)
    def _():
        tsum_ref[...] = part

    @pl.when(i > 0)
    def _():
        tsum_ref[...] = tsum_ref[...] + part


def _tc2a(sums, bg, ag, wat, ba):
    return pl.pallas_call(
        _tc2a_body,
        grid=(NB,),
        in_specs=[
            pl.BlockSpec((P, BN, D), lambda i: (0, i, 0)),
            pl.BlockSpec((P, 1, D), lambda i: (0, 0, 0)),
            pl.BlockSpec((P, 1, D), lambda i: (0, 0, 0)),
            pl.BlockSpec((D, D), lambda i: (0, 0)),
            pl.BlockSpec((1, D), lambda i: (0, 0)),
        ],
        out_specs=pl.BlockSpec((P, D), lambda i: (0, 0)),
        out_shape=jax.ShapeDtypeStruct((P, D), jnp.float32),
    )(sums, bg, ag, wat, ba)


def _tc2b_body(sums_ref, bg_ref, ag_ref, tsum_ref, att_ref, z_ref):
    e = _prelu(sums_ref[...], bg_ref[...], ag_ref[...])
    sp = tsum_ref[...] * (1.0 / N)
    logits = jnp.sum(sp * att_ref[...], axis=1)
    m = jnp.max(logits)
    w = jnp.exp(logits - m)
    beta = w / jnp.sum(w)
    z_ref[...] = beta[0] * e[0] + beta[1] * e[1]


def _tc2b(sums, bg, ag, tsum, att):
    return pl.pallas_call(
        _tc2b_body,
        grid=(NB,),
        in_specs=[
            pl.BlockSpec((P, BN, D), lambda i: (0, i, 0)),
            pl.BlockSpec((P, 1, D), lambda i: (0, 0, 0)),
            pl.BlockSpec((P, 1, D), lambda i: (0, 0, 0)),
            pl.BlockSpec((P, D), lambda i: (0, 0)),
            pl.BlockSpec((1, D), lambda i: (0, 0)),
        ],
        out_specs=pl.BlockSpec((BN, D), lambda i: (i, 0)),
        out_shape=jax.ShapeDtypeStruct((N, D), jnp.float32),
    )(sums, bg, ag, tsum, att)


def kernel(feats_0, adj0_indices, adj0_values, adj1_indices, adj1_values,
           W_feat, b_feat, W_gcn0, b_gcn0, a_gcn0, W_gcn1, b_gcn1, a_gcn1,
           W_attfc, b_attfc, att_vec):
    pad = E_PAD - E
    z_i = jnp.zeros((pad,), jnp.int32)
    z_f = jnp.zeros((pad,), jnp.float32)

    src0 = jnp.concatenate([adj0_indices[1].astype(jnp.int32), z_i])
    dst0 = jnp.concatenate([adj0_indices[0].astype(jnp.int32), z_i])
    val0 = jnp.concatenate([adj0_values, z_f])
    src1 = jnp.concatenate([adj1_indices[1].astype(jnp.int32), z_i]) + N
    dst1 = jnp.concatenate([adj1_indices[0].astype(jnp.int32), z_i])
    val1 = jnp.concatenate([adj1_values, z_f])

    srcs = jnp.stack([src0, src1]).reshape(P, NS, NSUP, SUP, CHUNK)
    dsts = jnp.stack([dst0, dst1]).reshape(P, NS, NSUP, SUP, CHUNK)
    vals = jnp.stack([val0, val1]).reshape(P, NS, NSUP, SUP, CHUNK)

    seq = _tc1(feats_0, W_feat.T, b_feat.reshape(1, D),
               W_gcn0.T, W_gcn1.T)

    sums = _sc_aggregate(seq.reshape(P * N, D), srcs, dsts, vals)
    sums = sums.reshape(P, N, D)

    bg = jnp.stack([b_gcn0, b_gcn1]).reshape(P, 1, D)
    ag = jnp.stack([jnp.broadcast_to(a_gcn0, (D,)),
                    jnp.broadcast_to(a_gcn1, (D,))]).reshape(P, 1, D)
    tsum = _tc2a(sums, bg, ag, W_attfc.T, b_attfc.reshape(1, D))
    return _tc2b(sums, bg, ag, tsum, att_vec)


# speedup vs baseline: 1.1112x; 1.1112x over previous
import functools

import jax
import jax.numpy as jnp
from jax import lax
from jax.experimental import pallas as pl
from jax.experimental.pallas import tpu as pltpu
from jax.experimental.pallas import tpu_sc as plsc

N = 10000
E = 320000
D = 128
P = 2

NC = 2
NS = 16
CHUNK = 128
SUP = 16
NSUP = -(-E // (NS * SUP * CHUNK))
TPT = NSUP * SUP * CHUNK
E_PAD = NS * TPT

BN = 1000
NB = N // BN


def _tc1_body(feats_ref, wft_ref, bf_ref, w0t_ref, w1t_ref, out_ref):
    x = feats_ref[...]
    h = jnp.dot(x, wft_ref[...], preferred_element_type=jnp.float32)
    h = h + bf_ref[...]
    h = jnp.where(h > 0, h, jnp.exp(jnp.minimum(h, 0.0)) - 1.0)
    out_ref[0] = jnp.dot(h, w0t_ref[...], preferred_element_type=jnp.float32)
    out_ref[1] = jnp.dot(h, w1t_ref[...], preferred_element_type=jnp.float32)


def _tc1(feats, wft, bf, w0t, w1t):
    return pl.pallas_call(
        _tc1_body,
        grid=(NB,),
        in_specs=[
            pl.BlockSpec((BN, D), lambda i: (i, 0)),
            pl.BlockSpec((D, D), lambda i: (0, 0)),
            pl.BlockSpec((1, D), lambda i: (0, 0)),
            pl.BlockSpec((D, D), lambda i: (0, 0)),
            pl.BlockSpec((D, D), lambda i: (0, 0)),
        ],
        out_specs=pl.BlockSpec((P, BN, D), lambda i: (0, i, 0)),
        out_shape=jax.ShapeDtypeStruct((P, N, D), jnp.float32),
    )(feats, wft, bf, w0t, w1t)


def _sc_body(seq_hbm, src_hbm, dst_hbm, val_hbm, out_hbm,
             idx_src, idx_dst, vals_v, rows, acc, sem, sem2, ssem):
    cid = lax.axis_index("c")
    sid = lax.axis_index("s")

    def _zrow(r, _):
        for j in range(D // 16):
            rows[0, r, pl.ds(j * 16, 16)] = jnp.zeros((16,), jnp.float32)
        return 0
    lax.fori_loop(0, CHUNK, _zrow, 0)

    base = sid * 624
    for i in range(4):
        pltpu.sync_copy(rows.at[0], acc.at[pl.ds(base + i * 128, 128)])
    pltpu.sync_copy(rows.at[0, pl.ds(0, 112)],
                    acc.at[pl.ds(base + 512, 112)])

    @pl.when(sid == 0)
    def _():
        pltpu.sync_copy(rows.at[0, pl.ds(0, 16)],
                        acc.at[pl.ds(NS * 624, 16)])
    plsc.subcore_barrier()

    def _scale(slot, c):
        def _grp(g, _):
            vv = vals_v[c, pl.ds(g * 16, 16)]
            for k in range(16):
                e = g * 16 + k
                v = vv[k]
                for j in range(D // 16):
                    sl = pl.ds(j * 16, 16)
                    rows[slot, e, sl] = rows[slot, e, sl] * v
            return 0
        lax.fori_loop(0, CHUNK // 16, _grp, 0)

    def _gather_start(c, slot, gsem):
        pltpu.async_copy(seq_hbm.at[idx_src.at[c]], rows.at[slot], gsem)

    def _gather_wait(c, slot, gsem):
        pltpu.make_async_copy(seq_hbm.at[idx_src.at[c]], rows.at[slot],
                              gsem).wait()

    def _sup(s, _):
        pltpu.sync_copy(src_hbm.at[cid, sid, s], idx_src)
        pltpu.sync_copy(dst_hbm.at[cid, sid, s], idx_dst)
        pltpu.sync_copy(val_hbm.at[cid, sid, s], vals_v)

        _gather_start(0, 0, sem)

        def _step(j, _):
            c0 = 2 * j
            c1 = 2 * j + 1
            _gather_wait(c0, 0, sem)
            _gather_start(c1, 1, sem2)
            _scale(0, c0)
            pltpu.sync_copy(rows.at[0], acc.at[idx_dst.at[c0]], add=True)

            _gather_wait(c1, 1, sem2)

            @pl.when(j < SUP // 2 - 1)
            def _():
                _gather_start(c1 + 1, 0, sem)
            _scale(1, c1)
            pltpu.sync_copy(rows.at[1], acc.at[idx_dst.at[c1]], add=True)
            return 0

        lax.fori_loop(0, SUP // 2, _step, 0)
        return 0

    lax.fori_loop(0, NSUP, _sup, 0)
    plsc.subcore_barrier()

    out_base = cid * N + base
    pltpu.sync_copy(acc.at[pl.ds(base, 624)],
                    out_hbm.at[pl.ds(out_base, 624)])

    @pl.when(sid == 0)
    def _():
        pltpu.sync_copy(acc.at[pl.ds(NS * 624, 16)],
                        out_hbm.at[pl.ds(cid * N + NS * 624, 16)])


@functools.partial(
    pl.kernel,
    out_type=jax.ShapeDtypeStruct((P * N, D), jnp.float32),
    mesh=plsc.VectorSubcoreMesh(core_axis_name="c", subcore_axis_name="s"),
    scratch_types=[
        pltpu.VMEM((SUP, CHUNK), jnp.int32),
        pltpu.VMEM((SUP, CHUNK), jnp.int32),
        pltpu.VMEM((SUP, CHUNK), jnp.float32),
        pltpu.VMEM((2, CHUNK, D), jnp.float32),
        pltpu.VMEM_SHARED((N, D), jnp.float32),
        pltpu.SemaphoreType.DMA,
        pltpu.SemaphoreType.DMA,
        pltpu.SemaphoreType.DMA,
    ],
)
def _sc_aggregate(seq_hbm, src_hbm, dst_hbm, val_hbm, out_hbm,
                  idx_src, idx_dst, vals_v, rows, acc, sem, sem2, ssem):
    _sc_body(seq_hbm, src_hbm, dst_hbm, val_hbm, out_hbm,
             idx_src, idx_dst, vals_v, rows, acc, sem, sem2, ssem)


def _tc2a_body(sums_ref, bg_ref, ag_ref, wat_ref, ba_ref, e_ref, tsum_ref):
    i = pl.program_id(0)
    x = sums_ref[...] + bg_ref[...]
    e = jnp.where(x >= 0, x, ag_ref[...] * x)
    e_ref[...] = e
    t0 = jnp.tanh(jnp.dot(e[0], wat_ref[...],
                          preferred_element_type=jnp.float32) + ba_ref[...])
    t1 = jnp.tanh(jnp.dot(e[1], wat_ref[...],
                          preferred_element_type=jnp.float32) + ba_ref[...])
    part = jnp.stack([jnp.sum(t0, axis=0), jnp.sum(t1, axis=0)])

    @pl.when(i == 0)
    def _():
        tsum_ref[...] = part

    @pl.when(i > 0)
    def _():
        tsum_ref[...] = tsum_ref[...] + part


def _tc2a(sums, bg, ag, wat, ba):
    return pl.pallas_call(
        _tc2a_body,
        grid=(NB,),
        in_specs=[
            pl.BlockSpec((P, BN, D), lambda i: (0, i, 0)),
            pl.BlockSpec((P, 1, D), lambda i: (0, 0, 0)),
            pl.BlockSpec((P, 1, D), lambda i: (0, 0, 0)),
            pl.BlockSpec((D, D), lambda i: (0, 0)),
            pl.BlockSpec((1, D), lambda i: (0, 0)),
        ],
        out_specs=[
            pl.BlockSpec((P, BN, D), lambda i: (0, i, 0)),
            pl.BlockSpec((P, D), lambda i: (0, 0)),
        ],
        out_shape=[
            jax.ShapeDtypeStruct((P, N, D), jnp.float32),
            jax.ShapeDtypeStruct((P, D), jnp.float32),
        ],
    )(sums, bg, ag, wat, ba)


def _tc2b_body(e_ref, tsum_ref, att_ref, z_ref):
    sp = tsum_ref[...] * (1.0 / N)
    logits = jnp.sum(sp * att_ref[...], axis=1)
    m = jnp.max(logits)
    w = jnp.exp(logits - m)
    beta = w / jnp.sum(w)
    z_ref[...] = beta[0] * e_ref[0] + beta[1] * e_ref[1]


def _tc2b(e, tsum, att):
    return pl.pallas_call(
        _tc2b_body,
        grid=(NB,),
        in_specs=[
            pl.BlockSpec((P, BN, D), lambda i: (0, i, 0)),
            pl.BlockSpec((P, D), lambda i: (0, 0)),
            pl.BlockSpec((1, D), lambda i: (0, 0)),
        ],
        out_specs=pl.BlockSpec((BN, D), lambda i: (i, 0)),
        out_shape=jax.ShapeDtypeStruct((N, D), jnp.float32),
    )(e, tsum, att)


def kernel(feats_0, adj0_indices, adj0_values, adj1_indices, adj1_values,
           W_feat, b_feat, W_gcn0, b_gcn0, a_gcn0, W_gcn1, b_gcn1, a_gcn1,
           W_attfc, b_attfc, att_vec):
    pad = E_PAD - E
    z_i = jnp.zeros((pad,), jnp.int32)
    z_f = jnp.zeros((pad,), jnp.float32)

    src0 = jnp.concatenate([adj0_indices[1].astype(jnp.int32), z_i])
    dst0 = jnp.concatenate([adj0_indices[0].astype(jnp.int32), z_i])
    val0 = jnp.concatenate([adj0_values, z_f])
    src1 = jnp.concatenate([adj1_indices[1].astype(jnp.int32), z_i]) + N
    dst1 = jnp.concatenate([adj1_indices[0].astype(jnp.int32), z_i])
    val1 = jnp.concatenate([adj1_values, z_f])

    srcs = jnp.stack([src0, src1]).reshape(P, NS, NSUP, SUP, CHUNK)
    dsts = jnp.stack([dst0, dst1]).reshape(P, NS, NSUP, SUP, CHUNK)
    vals = jnp.stack([val0, val1]).reshape(P, NS, NSUP, SUP, CHUNK)

    seq = _tc1(feats_0, W_feat.T, b_feat.reshape(1, D),
               W_gcn0.T, W_gcn1.T)

    sums = _sc_aggregate(seq.reshape(P * N, D), srcs, dsts, vals)
    sums = sums.reshape(P, N, D)

    bg = jnp.stack([b_gcn0, b_gcn1]).reshape(P, 1, D)
    ag = jnp.stack([jnp.broadcast_to(a_gcn0, (D,)),
                    jnp.broadcast_to(a_gcn1, (D,))]).reshape(P, 1, D)
    e, tsum = _tc2a(sums, bg, ag, W_attfc.T, b_attfc.reshape(1, D))
    return _tc2b(e, tsum, att_vec)


# speedup vs baseline: 1.1264x; 1.0136x over previous
import functools

import jax
import jax.numpy as jnp
from jax import lax
from jax.experimental import pallas as pl
from jax.experimental.pallas import tpu as pltpu
from jax.experimental.pallas import tpu_sc as plsc

N = 10000
E = 320000
D = 128
P = 2

NC = 2
NS = 16
CHUNK = 128
SUP = 32
NSUP = -(-E // (NS * SUP * CHUNK))
TPT = NSUP * SUP * CHUNK
E_PAD = NS * TPT

BN = 1000
NB = N // BN


def _tc1_body(feats_ref, wft_ref, bf_ref, w0t_ref, w1t_ref, out_ref):
    x = feats_ref[...]
    h = jnp.dot(x, wft_ref[...], preferred_element_type=jnp.float32)
    h = h + bf_ref[...]
    h = jnp.where(h > 0, h, jnp.exp(jnp.minimum(h, 0.0)) - 1.0)
    out_ref[0] = jnp.dot(h, w0t_ref[...], preferred_element_type=jnp.float32)
    out_ref[1] = jnp.dot(h, w1t_ref[...], preferred_element_type=jnp.float32)


def _tc1(feats, wft, bf, w0t, w1t):
    return pl.pallas_call(
        _tc1_body,
        grid=(NB,),
        in_specs=[
            pl.BlockSpec((BN, D), lambda i: (i, 0)),
            pl.BlockSpec((D, D), lambda i: (0, 0)),
            pl.BlockSpec((1, D), lambda i: (0, 0)),
            pl.BlockSpec((D, D), lambda i: (0, 0)),
            pl.BlockSpec((D, D), lambda i: (0, 0)),
        ],
        out_specs=pl.BlockSpec((P, BN, D), lambda i: (0, i, 0)),
        out_shape=jax.ShapeDtypeStruct((P, N, D), jnp.float32),
    )(feats, wft, bf, w0t, w1t)


def _sc_body(seq_hbm, src_hbm, dst_hbm, val_hbm, out_hbm,
             idx_src, idx_dst, vals_v, rows, acc, sem, sem2, ssem):
    cid = lax.axis_index("c")
    sid = lax.axis_index("s")

    def _zrow(r, _):
        for j in range(D // 16):
            rows[0, r, pl.ds(j * 16, 16)] = jnp.zeros((16,), jnp.float32)
        return 0
    lax.fori_loop(0, CHUNK, _zrow, 0)

    base = sid * 624
    for i in range(4):
        pltpu.sync_copy(rows.at[0], acc.at[pl.ds(base + i * 128, 128)])
    pltpu.sync_copy(rows.at[0, pl.ds(0, 112)],
                    acc.at[pl.ds(base + 512, 112)])

    @pl.when(sid == 0)
    def _():
        pltpu.sync_copy(rows.at[0, pl.ds(0, 16)],
                        acc.at[pl.ds(NS * 624, 16)])
    plsc.subcore_barrier()

    def _scale(slot, c):
        def _grp(g, _):
            vv = vals_v[c, pl.ds(g * 16, 16)]
            for k in range(16):
                e = g * 16 + k
                v = vv[k]
                for j in range(D // 16):
                    sl = pl.ds(j * 16, 16)
                    rows[slot, e, sl] = rows[slot, e, sl] * v
            return 0
        lax.fori_loop(0, CHUNK // 16, _grp, 0)

    def _gather_start(c, slot, gsem):
        pltpu.async_copy(seq_hbm.at[idx_src.at[c]], rows.at[slot], gsem)

    def _gather_wait(c, slot, gsem):
        pltpu.make_async_copy(seq_hbm.at[idx_src.at[c]], rows.at[slot],
                              gsem).wait()

    def _sup(s, _):
        pltpu.sync_copy(src_hbm.at[cid, sid, s], idx_src)
        pltpu.sync_copy(dst_hbm.at[cid, sid, s], idx_dst)
        pltpu.sync_copy(val_hbm.at[cid, sid, s], vals_v)

        _gather_start(0, 0, sem)

        def _step(j, _):
            c0 = 2 * j
            c1 = 2 * j + 1
            _gather_wait(c0, 0, sem)
            _gather_start(c1, 1, sem2)
            _scale(0, c0)
            pltpu.sync_copy(rows.at[0], acc.at[idx_dst.at[c0]], add=True)

            _gather_wait(c1, 1, sem2)

            @pl.when(j < SUP // 2 - 1)
            def _():
                _gather_start(c1 + 1, 0, sem)
            _scale(1, c1)
            pltpu.sync_copy(rows.at[1], acc.at[idx_dst.at[c1]], add=True)
            return 0

        lax.fori_loop(0, SUP // 2, _step, 0)
        return 0

    lax.fori_loop(0, NSUP, _sup, 0)
    plsc.subcore_barrier()

    out_base = cid * N + base
    pltpu.sync_copy(acc.at[pl.ds(base, 624)],
                    out_hbm.at[pl.ds(out_base, 624)])

    @pl.when(sid == 0)
    def _():
        pltpu.sync_copy(acc.at[pl.ds(NS * 624, 16)],
                        out_hbm.at[pl.ds(cid * N + NS * 624, 16)])


@functools.partial(
    pl.kernel,
    out_type=jax.ShapeDtypeStruct((P * N, D), jnp.float32),
    mesh=plsc.VectorSubcoreMesh(core_axis_name="c", subcore_axis_name="s"),
    scratch_types=[
        pltpu.VMEM((SUP, CHUNK), jnp.int32),
        pltpu.VMEM((SUP, CHUNK), jnp.int32),
        pltpu.VMEM((SUP, CHUNK), jnp.float32),
        pltpu.VMEM((2, CHUNK, D), jnp.float32),
        pltpu.VMEM_SHARED((N, D), jnp.float32),
        pltpu.SemaphoreType.DMA,
        pltpu.SemaphoreType.DMA,
        pltpu.SemaphoreType.DMA,
    ],
)
def _sc_aggregate(seq_hbm, src_hbm, dst_hbm, val_hbm, out_hbm,
                  idx_src, idx_dst, vals_v, rows, acc, sem, sem2, ssem):
    _sc_body(seq_hbm, src_hbm, dst_hbm, val_hbm, out_hbm,
             idx_src, idx_dst, vals_v, rows, acc, sem, sem2, ssem)


def _tc2a_body(sums_ref, bg_ref, ag_ref, wat_ref, ba_ref, e_ref, tsum_ref):
    i = pl.program_id(0)
    x = sums_ref[...] + bg_ref[...]
    e = jnp.where(x >= 0, x, ag_ref[...] * x)
    e_ref[...] = e
    t0 = jnp.tanh(jnp.dot(e[0], wat_ref[...],
                          preferred_element_type=jnp.float32) + ba_ref[...])
    t1 = jnp.tanh(jnp.dot(e[1], wat_ref[...],
                          preferred_element_type=jnp.float32) + ba_ref[...])
    part = jnp.stack([jnp.sum(t0, axis=0), jnp.sum(t1, axis=0)])

    @pl.when(i == 0)
    def _():
        tsum_ref[...] = part

    @pl.when(i > 0)
    def _():
        tsum_ref[...] = tsum_ref[...] + part


def _tc2a(sums, bg, ag, wat, ba):
    return pl.pallas_call(
        _tc2a_body,
        grid=(NB,),
        in_specs=[
            pl.BlockSpec((P, BN, D), lambda i: (0, i, 0)),
            pl.BlockSpec((P, 1, D), lambda i: (0, 0, 0)),
            pl.BlockSpec((P, 1, D), lambda i: (0, 0, 0)),
            pl.BlockSpec((D, D), lambda i: (0, 0)),
            pl.BlockSpec((1, D), lambda i: (0, 0)),
        ],
        out_specs=[
            pl.BlockSpec((P, BN, D), lambda i: (0, i, 0)),
            pl.BlockSpec((P, D), lambda i: (0, 0)),
        ],
        out_shape=[
            jax.ShapeDtypeStruct((P, N, D), jnp.float32),
            jax.ShapeDtypeStruct((P, D), jnp.float32),
        ],
    )(sums, bg, ag, wat, ba)


def _tc2b_body(e_ref, tsum_ref, att_ref, z_ref):
    sp = tsum_ref[...] * (1.0 / N)
    logits = jnp.sum(sp * att_ref[...], axis=1)
    m = jnp.max(logits)
    w = jnp.exp(logits - m)
    beta = w / jnp.sum(w)
    z_ref[...] = beta[0] * e_ref[0] + beta[1] * e_ref[1]


def _tc2b(e, tsum, att):
    return pl.pallas_call(
        _tc2b_body,
        grid=(NB,),
        in_specs=[
            pl.BlockSpec((P, BN, D), lambda i: (0, i, 0)),
            pl.BlockSpec((P, D), lambda i: (0, 0)),
            pl.BlockSpec((1, D), lambda i: (0, 0)),
        ],
        out_specs=pl.BlockSpec((BN, D), lambda i: (i, 0)),
        out_shape=jax.ShapeDtypeStruct((N, D), jnp.float32),
    )(e, tsum, att)


def kernel(feats_0, adj0_indices, adj0_values, adj1_indices, adj1_values,
           W_feat, b_feat, W_gcn0, b_gcn0, a_gcn0, W_gcn1, b_gcn1, a_gcn1,
           W_attfc, b_attfc, att_vec):
    pad = E_PAD - E
    z_i = jnp.zeros((pad,), jnp.int32)
    z_f = jnp.zeros((pad,), jnp.float32)

    src0 = jnp.concatenate([adj0_indices[1].astype(jnp.int32), z_i])
    dst0 = jnp.concatenate([adj0_indices[0].astype(jnp.int32), z_i])
    val0 = jnp.concatenate([adj0_values, z_f])
    src1 = jnp.concatenate([adj1_indices[1].astype(jnp.int32), z_i]) + N
    dst1 = jnp.concatenate([adj1_indices[0].astype(jnp.int32), z_i])
    val1 = jnp.concatenate([adj1_values, z_f])

    srcs = jnp.stack([src0, src1]).reshape(P, NS, NSUP, SUP, CHUNK)
    dsts = jnp.stack([dst0, dst1]).reshape(P, NS, NSUP, SUP, CHUNK)
    vals = jnp.stack([val0, val1]).reshape(P, NS, NSUP, SUP, CHUNK)

    seq = _tc1(feats_0, W_feat.T, b_feat.reshape(1, D),
               W_gcn0.T, W_gcn1.T)

    sums = _sc_aggregate(seq.reshape(P * N, D), srcs, dsts, vals)
    sums = sums.reshape(P, N, D)

    bg = jnp.stack([b_gcn0, b_gcn1]).reshape(P, 1, D)
    ag = jnp.stack([jnp.broadcast_to(a_gcn0, (D,)),
                    jnp.broadcast_to(a_gcn1, (D,))]).reshape(P, 1, D)
    e, tsum = _tc2a(sums, bg, ag, W_attfc.T, b_attfc.reshape(1, D))
    return _tc2b(e, tsum, att_vec)


# speedup vs baseline: 1.1366x; 1.0091x over previous
import functools

import jax
import jax.numpy as jnp
from jax import lax
from jax.experimental import pallas as pl
from jax.experimental.pallas import tpu as pltpu
from jax.experimental.pallas import tpu_sc as plsc

N = 10000
E = 320000
D = 128
P = 2

NC = 2
NS = 16
CHUNK = 128
SUP = 40
NSUP = -(-E // (NS * SUP * CHUNK))
TPT = NSUP * SUP * CHUNK
E_PAD = NS * TPT

BN = 1000
NB = N // BN


def _tc1_body(feats_ref, wft_ref, bf_ref, w0t_ref, w1t_ref, out_ref):
    x = feats_ref[...]
    h = jnp.dot(x, wft_ref[...], preferred_element_type=jnp.float32)
    h = h + bf_ref[...]
    h = jnp.where(h > 0, h, jnp.exp(jnp.minimum(h, 0.0)) - 1.0)
    out_ref[0] = jnp.dot(h, w0t_ref[...], preferred_element_type=jnp.float32)
    out_ref[1] = jnp.dot(h, w1t_ref[...], preferred_element_type=jnp.float32)


def _tc1(feats, wft, bf, w0t, w1t):
    return pl.pallas_call(
        _tc1_body,
        grid=(NB,),
        in_specs=[
            pl.BlockSpec((BN, D), lambda i: (i, 0)),
            pl.BlockSpec((D, D), lambda i: (0, 0)),
            pl.BlockSpec((1, D), lambda i: (0, 0)),
            pl.BlockSpec((D, D), lambda i: (0, 0)),
            pl.BlockSpec((D, D), lambda i: (0, 0)),
        ],
        out_specs=pl.BlockSpec((P, BN, D), lambda i: (0, i, 0)),
        out_shape=jax.ShapeDtypeStruct((P, N, D), jnp.float32),
    )(feats, wft, bf, w0t, w1t)


def _sc_body(seq_hbm, src_hbm, dst_hbm, val_hbm, out_hbm,
             idx_src, idx_dst, vals_v, rows, acc, sem, sem2, ssem):
    cid = lax.axis_index("c")
    sid = lax.axis_index("s")

    def _zrow(r, _):
        for j in range(D // 16):
            rows[0, r, pl.ds(j * 16, 16)] = jnp.zeros((16,), jnp.float32)
        return 0
    lax.fori_loop(0, CHUNK, _zrow, 0)

    base = sid * 624
    for i in range(4):
        pltpu.sync_copy(rows.at[0], acc.at[pl.ds(base + i * 128, 128)])
    pltpu.sync_copy(rows.at[0, pl.ds(0, 112)],
                    acc.at[pl.ds(base + 512, 112)])

    @pl.when(sid == 0)
    def _():
        pltpu.sync_copy(rows.at[0, pl.ds(0, 16)],
                        acc.at[pl.ds(NS * 624, 16)])
    plsc.subcore_barrier()

    def _scale(slot, c):
        def _grp(g, _):
            vv = vals_v[c, pl.ds(g * 16, 16)]
            for k in range(16):
                e = g * 16 + k
                v = vv[k]
                for j in range(D // 16):
                    sl = pl.ds(j * 16, 16)
                    rows[slot, e, sl] = rows[slot, e, sl] * v
            return 0
        lax.fori_loop(0, CHUNK // 16, _grp, 0)

    def _gather_start(c, slot, gsem):
        pltpu.async_copy(seq_hbm.at[idx_src.at[c]], rows.at[slot], gsem)

    def _gather_wait(c, slot, gsem):
        pltpu.make_async_copy(seq_hbm.at[idx_src.at[c]], rows.at[slot],
                              gsem).wait()

    def _sup(s, _):
        pltpu.sync_copy(src_hbm.at[cid, sid, s], idx_src)
        pltpu.sync_copy(dst_hbm.at[cid, sid, s], idx_dst)
        pltpu.sync_copy(val_hbm.at[cid, sid, s], vals_v)

        _gather_start(0, 0, sem)

        def _step(j, _):
            c0 = 2 * j
            c1 = 2 * j + 1
            _gather_wait(c0, 0, sem)
            _gather_start(c1, 1, sem2)
            _scale(0, c0)
            pltpu.sync_copy(rows.at[0], acc.at[idx_dst.at[c0]], add=True)

            _gather_wait(c1, 1, sem2)

            @pl.when(j < SUP // 2 - 1)
            def _():
                _gather_start(c1 + 1, 0, sem)
            _scale(1, c1)
            pltpu.sync_copy(rows.at[1], acc.at[idx_dst.at[c1]], add=True)
            return 0

        lax.fori_loop(0, SUP // 2, _step, 0)
        return 0

    lax.fori_loop(0, NSUP, _sup, 0)
    plsc.subcore_barrier()

    out_base = cid * N + base
    pltpu.sync_copy(acc.at[pl.ds(base, 624)],
                    out_hbm.at[pl.ds(out_base, 624)])

    @pl.when(sid == 0)
    def _():
        pltpu.sync_copy(acc.at[pl.ds(NS * 624, 16)],
                        out_hbm.at[pl.ds(cid * N + NS * 624, 16)])


@functools.partial(
    pl.kernel,
    out_type=jax.ShapeDtypeStruct((P * N, D), jnp.float32),
    mesh=plsc.VectorSubcoreMesh(core_axis_name="c", subcore_axis_name="s"),
    scratch_types=[
        pltpu.VMEM((SUP, CHUNK), jnp.int32),
        pltpu.VMEM((SUP, CHUNK), jnp.int32),
        pltpu.VMEM((SUP, CHUNK), jnp.float32),
        pltpu.VMEM((2, CHUNK, D), jnp.float32),
        pltpu.VMEM_SHARED((N, D), jnp.float32),
        pltpu.SemaphoreType.DMA,
        pltpu.SemaphoreType.DMA,
        pltpu.SemaphoreType.DMA,
    ],
)
def _sc_aggregate(seq_hbm, src_hbm, dst_hbm, val_hbm, out_hbm,
                  idx_src, idx_dst, vals_v, rows, acc, sem, sem2, ssem):
    _sc_body(seq_hbm, src_hbm, dst_hbm, val_hbm, out_hbm,
             idx_src, idx_dst, vals_v, rows, acc, sem, sem2, ssem)


def _tc2a_body(sums_ref, bg_ref, ag_ref, wat_ref, ba_ref, e_ref, tsum_ref):
    i = pl.program_id(0)
    x = sums_ref[...] + bg_ref[...]
    e = jnp.where(x >= 0, x, ag_ref[...] * x)
    e_ref[...] = e
    t0 = jnp.tanh(jnp.dot(e[0], wat_ref[...],
                          preferred_element_type=jnp.float32) + ba_ref[...])
    t1 = jnp.tanh(jnp.dot(e[1], wat_ref[...],
                          preferred_element_type=jnp.float32) + ba_ref[...])
    part = jnp.stack([jnp.sum(t0, axis=0), jnp.sum(t1, axis=0)])

    @pl.when(i == 0)
    def _():
        tsum_ref[...] = part

    @pl.when(i > 0)
    def _():
        tsum_ref[...] = tsum_ref[...] + part


def _tc2a(sums, bg, ag, wat, ba):
    return pl.pallas_call(
        _tc2a_body,
        grid=(NB,),
        in_specs=[
            pl.BlockSpec((P, BN, D), lambda i: (0, i, 0)),
            pl.BlockSpec((P, 1, D), lambda i: (0, 0, 0)),
            pl.BlockSpec((P, 1, D), lambda i: (0, 0, 0)),
            pl.BlockSpec((D, D), lambda i: (0, 0)),
            pl.BlockSpec((1, D), lambda i: (0, 0)),
        ],
        out_specs=[
            pl.BlockSpec((P, BN, D), lambda i: (0, i, 0)),
            pl.BlockSpec((P, D), lambda i: (0, 0)),
        ],
        out_shape=[
            jax.ShapeDtypeStruct((P, N, D), jnp.float32),
            jax.ShapeDtypeStruct((P, D), jnp.float32),
        ],
    )(sums, bg, ag, wat, ba)


def _tc2b_body(e_ref, tsum_ref, att_ref, z_ref):
    sp = tsum_ref[...] * (1.0 / N)
    logits = jnp.sum(sp * att_ref[...], axis=1)
    m = jnp.max(logits)
    w = jnp.exp(logits - m)
    beta = w / jnp.sum(w)
    z_ref[...] = beta[0] * e_ref[0] + beta[1] * e_ref[1]


def _tc2b(e, tsum, att):
    return pl.pallas_call(
        _tc2b_body,
        grid=(NB,),
        in_specs=[
            pl.BlockSpec((P, BN, D), lambda i: (0, i, 0)),
            pl.BlockSpec((P, D), lambda i: (0, 0)),
            pl.BlockSpec((1, D), lambda i: (0, 0)),
        ],
        out_specs=pl.BlockSpec((BN, D), lambda i: (i, 0)),
        out_shape=jax.ShapeDtypeStruct((N, D), jnp.float32),
    )(e, tsum, att)


def kernel(feats_0, adj0_indices, adj0_values, adj1_indices, adj1_values,
           W_feat, b_feat, W_gcn0, b_gcn0, a_gcn0, W_gcn1, b_gcn1, a_gcn1,
           W_attfc, b_attfc, att_vec):
    pad = E_PAD - E
    z_i = jnp.zeros((pad,), jnp.int32)
    z_f = jnp.zeros((pad,), jnp.float32)

    src0 = jnp.concatenate([adj0_indices[1].astype(jnp.int32), z_i])
    dst0 = jnp.concatenate([adj0_indices[0].astype(jnp.int32), z_i])
    val0 = jnp.concatenate([adj0_values, z_f])
    src1 = jnp.concatenate([adj1_indices[1].astype(jnp.int32), z_i]) + N
    dst1 = jnp.concatenate([adj1_indices[0].astype(jnp.int32), z_i])
    val1 = jnp.concatenate([adj1_values, z_f])

    srcs = jnp.stack([src0, src1]).reshape(P, NS, NSUP, SUP, CHUNK)
    dsts = jnp.stack([dst0, dst1]).reshape(P, NS, NSUP, SUP, CHUNK)
    vals = jnp.stack([val0, val1]).reshape(P, NS, NSUP, SUP, CHUNK)

    seq = _tc1(feats_0, W_feat.T, b_feat.reshape(1, D),
               W_gcn0.T, W_gcn1.T)

    sums = _sc_aggregate(seq.reshape(P * N, D), srcs, dsts, vals)
    sums = sums.reshape(P, N, D)

    bg = jnp.stack([b_gcn0, b_gcn1]).reshape(P, 1, D)
    ag = jnp.stack([jnp.broadcast_to(a_gcn0, (D,)),
                    jnp.broadcast_to(a_gcn1, (D,))]).reshape(P, 1, D)
    e, tsum = _tc2a(sums, bg, ag, W_attfc.T, b_attfc.reshape(1, D))
    return _tc2b(e, tsum, att_vec)
